# Initial kernel scaffold; baseline (speedup 1.0000x reference)
import jax
import jax.numpy as jnp
from jax.experimental import pallas as pl


def kernel(xyz, feat):
    raise NotImplementedError("write your pallas kernel here")


# speedup vs baseline: 41.0741x; 41.0741x over previous
import numpy as np

import jax
import jax.numpy as jnp
from jax import lax
from jax.experimental import pallas as pl
from jax.experimental.pallas import tpu as pltpu
from jax.experimental.pallas import tpu_sc as plsc

_NC = 2
_NS = 16
_L = 16
_NW = _NC * _NS

_B = 8
_N = 4096
_D = 64
_K = 1024
_NB = 32
_R2 = np.float32(0.04)
_CHUNKS = _N // _L

_PPB = _NW // _B
_QPP = _K // _PPB
_GW = 128


def _fps_body(xyzt_hbm, aux_hbm, samp_hbm,
              xs_v, ys_v, zs_v, dists_v, sx_v, sy_v, sz_v, aux_v):
    c = lax.axis_index("c")
    s = lax.axis_index("s")
    w = s * _NC + c

    @pl.when(w < _B)
    def _():
        b = w
        pltpu.sync_copy(xyzt_hbm.at[b, 0], xs_v)
        pltpu.sync_copy(xyzt_hbm.at[b, 1], ys_v)
        pltpu.sync_copy(xyzt_hbm.at[b, 2], zs_v)
        pltpu.sync_copy(aux_hbm.at[b], aux_v)

        lane = jnp.arange(_L, dtype=jnp.int32)
        lane0 = lane == 0
        zeros = jnp.zeros((_L,), jnp.int32)

        inf_v = jnp.full((_L,), jnp.inf, jnp.float32)

        def init_chunk(ci, _):
            dists_v[pl.ds(ci * _L, _L)] = inf_v
            return 0

        lax.fori_loop(0, _CHUNKS, init_chunk, 0)

        qx0 = aux_v[pl.ds(0, _L)]
        qy0 = aux_v[pl.ds(_L, _L)]
        qz0 = aux_v[pl.ds(2 * _L, _L)]
        plsc.store_scatter(sx_v, [zeros], qx0, mask=lane0)
        plsc.store_scatter(sy_v, [zeros], qy0, mask=lane0)
        plsc.store_scatter(sz_v, [zeros], qz0, mask=lane0)

        def iter_body(i, carry):
            qx, qy, qz = carry

            def chunk(ci, st):
                bv, bi = st
                sl = pl.ds(ci * _L, _L)
                x = xs_v[sl]
                y = ys_v[sl]
                z = zs_v[sl]
                dx = x - qx
                dy = y - qy
                dz = z - qz
                d = (dx * dx + dz * dz) + dy * dy
                nd = jnp.minimum(dists_v[sl], d)
                dists_v[sl] = nd
                iv = ci * _L + lane
                m = nd > bv
                return jnp.where(m, nd, bv), jnp.where(m, iv, bi)

            bv, bi = lax.fori_loop(
                0, _CHUNKS, chunk,
                (jnp.full((_L,), -jnp.inf, jnp.float32),
                 jnp.zeros((_L,), jnp.int32)))
            gmax = jnp.max(bv)
            cand = jnp.where(bv == gmax, bi, jnp.int32(_N))
            widx = jnp.min(cand)
            ws = jnp.full((_L,), widx, jnp.int32)
            nqx = plsc.load_gather(xs_v, [ws])
            nqy = plsc.load_gather(ys_v, [ws])
            nqz = plsc.load_gather(zs_v, [ws])
            iw = jnp.full((_L,), i, jnp.int32)
            plsc.store_scatter(sx_v, [iw], nqx, mask=lane0)
            plsc.store_scatter(sy_v, [iw], nqy, mask=lane0)
            plsc.store_scatter(sz_v, [iw], nqz, mask=lane0)
            return nqx, nqy, nqz

        lax.fori_loop(1, _K, iter_body, (qx0, qy0, qz0))

        pltpu.sync_copy(sx_v, samp_hbm.at[b, 0])
        pltpu.sync_copy(sy_v, samp_hbm.at[b, 1])
        pltpu.sync_copy(sz_v, samp_hbm.at[b, 2])


def _bg_body(xyzt_hbm, samp_hbm, featf_hbm, out_hbm,
             xs_v, ys_v, zs_v, qx_v, qy_v, qz_v, rows_v, gbuf_v, gsem):
    c = lax.axis_index("c")
    s = lax.axis_index("s")
    w = s * _NC + c
    b = w // _PPB
    part = w % _PPB

    pltpu.sync_copy(xyzt_hbm.at[b, 0], xs_v)
    pltpu.sync_copy(xyzt_hbm.at[b, 1], ys_v)
    pltpu.sync_copy(xyzt_hbm.at[b, 2], zs_v)
    qoff = part * _QPP
    pltpu.sync_copy(samp_hbm.at[b, 0, pl.ds(qoff, _QPP)], qx_v)
    pltpu.sync_copy(samp_hbm.at[b, 1, pl.ds(qoff, _QPP)], qy_v)
    pltpu.sync_copy(samp_hbm.at[b, 2, pl.ds(qoff, _QPP)], qz_v)

    lane = jnp.arange(_L, dtype=jnp.int32)
    bbase = b * _N
    padv = jnp.full((_L,), bbase + _K, jnp.int32)

    def per_query(q, _):
        qsp = jnp.full((_L,), q, jnp.int32)
        qx = plsc.load_gather(qx_v, [qsp])
        qy = plsc.load_gather(qy_v, [qsp])
        qz = plsc.load_gather(qz_v, [qsp])
        base = q * _NB
        rows_v[pl.ds(base, _L)] = padv
        rows_v[pl.ds(base + _L, _L)] = padv

        def cond(st):
            return (st[0] < _CHUNKS) & (st[1] < _NB)

        def wbody(st):
            ci, cnt = st
            sl = pl.ds(ci * _L, _L)
            x = xs_v[sl]
            y = ys_v[sl]
            z = zs_v[sl]
            dx = x - qx
            dy = y - qy
            dz = z - qz
            d2 = (dx * dx + dz * dz) + dy * dy
            within = d2 < _R2
            hits = jnp.where(within, jnp.int32(1), jnp.int32(0))
            ranks = plsc.cumsum(hits) + (cnt - 1)
            valid = within & (ranks < _NB)
            gi = bbase + ci * _L + lane
            pos = jnp.where(valid, base + ranks, base + _NB - 1)
            plsc.store_scatter(rows_v, [pos], gi, mask=valid)
            return ci + 1, cnt + jnp.sum(hits)

        lax.while_loop(cond, wbody, (jnp.int32(0), jnp.int32(0)))
        return 0

    lax.fori_loop(0, _QPP, per_query, 0)

    obase = w * (_QPP * _NB)

    def gchunk(j, _):
        isl = rows_v.at[pl.ds(j * _GW, _GW)]
        pltpu.async_copy(featf_hbm.at[isl], gbuf_v, gsem).wait()
        pltpu.sync_copy(gbuf_v, out_hbm.at[pl.ds(obase + j * _GW, _GW)])
        return 0

    lax.fori_loop(0, (_QPP * _NB) // _GW, gchunk, 0)


@jax.jit
def kernel(xyz, feat):
    mesh = plsc.VectorSubcoreMesh(
        core_axis_name="c", subcore_axis_name="s",
        num_cores=_NC, num_subcores=_NS)

    xyz_t = jnp.transpose(xyz, (0, 2, 1))
    start = jax.random.randint(
        jax.random.key(42), (_B,), 0, _N).astype(jnp.int32)
    q0 = jnp.take_along_axis(xyz, start[:, None, None], axis=1)[:, 0, :]
    aux = jnp.repeat(q0, _L, axis=1)

    cparams = pltpu.CompilerParams(
        use_tc_tiling_on_sc=False, needs_layout_passes=False)
    fps_call = pl.kernel(
        _fps_body,
        out_type=jax.ShapeDtypeStruct((_B, 3, _K), jnp.float32),
        mesh=mesh,
        compiler_params=cparams,
        scratch_types=[
            pltpu.VMEM((_N,), jnp.float32),
            pltpu.VMEM((_N,), jnp.float32),
            pltpu.VMEM((_N,), jnp.float32),
            pltpu.VMEM((_N,), jnp.float32),
            pltpu.VMEM((_K,), jnp.float32),
            pltpu.VMEM((_K,), jnp.float32),
            pltpu.VMEM((_K,), jnp.float32),
            pltpu.VMEM((3 * _L,), jnp.float32),
        ],
    )
    samp = fps_call(xyz_t, aux)

    featf = feat.reshape(_B * _N, _D)
    bg_call = pl.kernel(
        _bg_body,
        out_type=jax.ShapeDtypeStruct((_B * _K * _NB, _D), jnp.float32),
        mesh=mesh,
        compiler_params=cparams,
        scratch_types=[
            pltpu.VMEM((_N,), jnp.float32),
            pltpu.VMEM((_N,), jnp.float32),
            pltpu.VMEM((_N,), jnp.float32),
            pltpu.VMEM((_QPP,), jnp.float32),
            pltpu.VMEM((_QPP,), jnp.float32),
            pltpu.VMEM((_QPP,), jnp.float32),
            pltpu.VMEM((_QPP * _NB,), jnp.int32),
            pltpu.VMEM((_GW, _D), jnp.float32),
            pltpu.SemaphoreType.DMA,
        ],
    )
    outf = bg_call(xyz_t, samp, featf)

    sample_xyz = jnp.transpose(samp, (0, 2, 1))
    neighbor_feat = outf.reshape(_B, _K, _NB, _D)
    return sample_xyz, neighbor_feat
